# baseline (speedup 1.0000x reference)
import functools

import jax
import jax.numpy as jnp
from jax import lax
from jax.experimental import pallas as pl
from jax.experimental.pallas import tpu as pltpu
from jax.experimental.pallas import tpu_sc as plsc

N = 10000
E = 320000
D = 128
DE = 16
NLAYER = 3
G = 512

NC, NS = 2, 16
NW = NC * NS
CH = 64
E_PAD = 327680
NCHUNK = E_PAD // CH
CPT = NCHUNK // NW
NBUF = 3
RPT = 624

NB = 1000
NBLK = N // NB
BE = 1280
NEB = E // BE
NEB_PAD = E_PAD // BE

NEG = -1e30


def _eemb_body(ea_ref, we_ref, e0_ref, e1_ref, e2_ref):
    i = pl.program_id(0)
    outs = (e0_ref, e1_ref, e2_ref)

    @pl.when(i < NEB)
    def _():
        ea = ea_ref[...]
        for l in range(NLAYER):
            outs[l][...] = jnp.dot(ea, we_ref[l],
                                   preferred_element_type=jnp.float32)

    @pl.when(i >= NEB)
    def _():
        for l in range(NLAYER):
            outs[l][...] = jnp.full((BE, D), NEG, jnp.float32)


def _eemb(edge_attr, w_edge):
    out = jax.ShapeDtypeStruct((E_PAD, D), jnp.float32)
    return pl.pallas_call(
        _eemb_body,
        grid=(NEB_PAD,),
        in_specs=[
            pl.BlockSpec((BE, DE), lambda i: (jnp.minimum(i, NEB - 1), 0)),
            pl.BlockSpec((NLAYER, DE, D), lambda i: (0, 0, 0)),
        ],
        out_specs=[
            pl.BlockSpec((BE, D), lambda i: (i, 0)),
            pl.BlockSpec((BE, D), lambda i: (i, 0)),
            pl.BlockSpec((BE, D), lambda i: (i, 0)),
        ],
        out_shape=[out, out, out],
    )(edge_attr, w_edge)


def _make_sc_msg():
    mesh = plsc.VectorSubcoreMesh(
        core_axis_name="c", subcore_axis_name="s", num_cores=NC, num_subcores=NS
    )

    @functools.partial(
        pl.kernel,
        out_type=jax.ShapeDtypeStruct((NC, N, D), jnp.float32),
        mesh=mesh,
        scratch_types=[
            pltpu.VMEM_SHARED((N, D), jnp.float32),
            pltpu.VMEM((NBUF, CH), jnp.int32),
            pltpu.VMEM((NBUF, CH), jnp.int32),
            pltpu.VMEM((NBUF, CH, D), jnp.float32),
            pltpu.VMEM((NBUF, CH, D), jnp.float32),
            pltpu.SemaphoreType.DMA((NBUF,)),
            pltpu.SemaphoreType.DMA((NBUF,)),
            pltpu.SemaphoreType.DMA((NBUF,)),
            pltpu.SemaphoreType.DMA((NBUF,)),
        ],
    )
    def sc_msg(h_hbm, e_hbm, src_hbm, dst_hbm, zeros_hbm, out_hbm,
               agg, srcring, dstring, rows, embs, isem, dsem, gsem, ssem):
        cid = lax.axis_index("c")
        sid = lax.axis_index("s")
        wid = sid * NC + cid
        base = wid * CPT
        r0 = sid * RPT

        pltpu.sync_copy(zeros_hbm.at[pl.ds(r0, RPT), :],
                        agg.at[pl.ds(r0, RPT), :])

        @pl.when(sid == NS - 1)
        def _():
            pltpu.sync_copy(zeros_hbm.at[pl.ds(RPT * NS, N - RPT * NS), :],
                            agg.at[pl.ds(RPT * NS, N - RPT * NS), :])

        plsc.subcore_barrier()

        def sidx_desc(c, b):
            return pltpu.make_async_copy(
                src_hbm.at[pl.ds((base + c) * CH, CH)], srcring.at[b],
                isem.at[b])

        def didx_desc(c, b):
            return pltpu.make_async_copy(
                dst_hbm.at[pl.ds((base + c) * CH, CH)], dstring.at[b],
                dsem.at[b])

        def gather_descs(c, b):
            return (
                pltpu.make_async_copy(h_hbm.at[srcring.at[b]], rows.at[b],
                                      gsem.at[b]),
                pltpu.make_async_copy(e_hbm.at[pl.ds((base + c) * CH, CH), :],
                                      embs.at[b], gsem.at[b]),
            )

        def scatter_desc(b):
            return pltpu.make_async_copy(rows.at[b], agg.at[dstring.at[b]],
                                         ssem.at[b])

        def issue_gather(c, b):
            dg, de = gather_descs(c, b)
            dg.start()
            de.start()

        def process(c, b, steady):
            dg, de = gather_descs(c, b)
            dg.wait()
            de.wait()

            if steady:
                @pl.when(c < CPT - 2)
                def _():
                    sidx_desc(c + 2, (c + 2) % NBUF).start()

            def row_body(r, carry):
                for q in range(D // 16):
                    sl = pl.ds(q * 16, 16)
                    rows[b, r, sl] = jnp.maximum(rows[b, r, sl] +
                                                 embs[b, r, sl], 0.0)
                return carry

            lax.fori_loop(0, CH, row_body, 0)

            didx_desc(c, b).wait()
            scatter_desc(b).start(add=True)

            if steady:
                b2 = (c + 2) % NBUF

                @pl.when(c >= 1)
                def _():
                    scatter_desc(b2).wait()

                @pl.when(c < CPT - 2)
                def _():
                    didx_desc(c + 2, b2).start()
                    sidx_desc(c + 2, b2).wait()
                    issue_gather(c + 2, b2)

        for c0 in range(2):
            sidx_desc(c0, c0).start()
            didx_desc(c0, c0).start()
        for c0 in range(2):
            sidx_desc(c0, c0).wait()
            issue_gather(c0, c0)

        def tri_body(g, carry):
            for boff in range(NBUF):
                c = g * NBUF + boff
                process(c, boff, True)
            return carry

        lax.fori_loop(0, (CPT - 1) // NBUF, tri_body, 0)
        process(CPT - 1, (CPT - 1) % NBUF, False)
        scatter_desc((CPT - 2) % NBUF).wait()
        scatter_desc((CPT - 1) % NBUF).wait()

        plsc.subcore_barrier()
        pltpu.sync_copy(agg.at[pl.ds(r0, RPT), :],
                        out_hbm.at[cid, pl.ds(r0, RPT), :])

        @pl.when(sid == NS - 1)
        def _():
            pltpu.sync_copy(agg.at[pl.ds(RPT * NS, N - RPT * NS), :],
                            out_hbm.at[cid, pl.ds(RPT * NS, N - RPT * NS), :])

    return sc_msg


def _mlp_body(last, parts_ref, h_ref, w1_ref, b1_ref, w2_ref, b2_ref,
              scale_ref, out_ref):
    t = parts_ref[0] + parts_ref[1] + scale_ref[0, 0] * h_ref[...]
    u = jnp.dot(t, w1_ref[...], preferred_element_type=jnp.float32)
    u = jnp.maximum(u + b1_ref[...], 0.0)
    v = jnp.dot(u, w2_ref[...], preferred_element_type=jnp.float32)
    v = v + b2_ref[...]
    if not last:
        v = jnp.maximum(v, 0.0)
    out_ref[...] = v


def _mlp(parts, h, w1, b1, w2, b2, scale, last):
    return pl.pallas_call(
        functools.partial(_mlp_body, last),
        grid=(NBLK,),
        in_specs=[
            pl.BlockSpec((NC, NB, D), lambda i: (0, i, 0)),
            pl.BlockSpec((NB, D), lambda i: (i, 0)),
            pl.BlockSpec((D, D), lambda i: (0, 0)),
            pl.BlockSpec((1, D), lambda i: (0, 0)),
            pl.BlockSpec((D, D), lambda i: (0, 0)),
            pl.BlockSpec((1, D), lambda i: (0, 0)),
            pl.BlockSpec((1, 1), lambda i: (0, 0)),
        ],
        out_specs=pl.BlockSpec((NB, D), lambda i: (i, 0)),
        out_shape=jax.ShapeDtypeStruct((N, D), jnp.float32),
    )(parts, h, w1, b1, w2, b2, scale)


def _pool_body(batch_ref, h_ref, out_ref, sums, counts):
    i = pl.program_id(0)
    b = batch_ref[0, 0, :]
    gid = lax.broadcasted_iota(jnp.int32, (G, NB), 0)
    mask = (b[None, :] == gid).astype(jnp.float32)
    psum = jnp.dot(mask, h_ref[...], preferred_element_type=jnp.float32)
    pcnt = jnp.broadcast_to(jnp.sum(mask, axis=1, keepdims=True), (G, D))

    @pl.when(i == 0)
    def _():
        sums[...] = psum
        counts[...] = pcnt

    @pl.when(i > 0)
    def _():
        sums[...] += psum
        counts[...] += pcnt

    @pl.when(i == NBLK - 1)
    def _():
        out_ref[...] = sums[...] / jnp.maximum(counts[...], 1.0)


def _pool(batch3d, h):
    return pl.pallas_call(
        _pool_body,
        grid=(NBLK,),
        in_specs=[
            pl.BlockSpec((1, 1, NB), lambda i: (i, 0, 0)),
            pl.BlockSpec((NB, D), lambda i: (i, 0)),
        ],
        out_specs=pl.BlockSpec((G, D), lambda i: (0, 0)),
        out_shape=jax.ShapeDtypeStruct((G, D), jnp.float32),
        scratch_shapes=[
            pltpu.VMEM((G, D), jnp.float32),
            pltpu.VMEM((G, D), jnp.float32),
        ],
    )(batch3d, h)


def kernel(x, edge_index, edge_attr, batch, W_edge, W1, b1, W2, b2, eps):
    pad = jnp.zeros((E_PAD - E,), jnp.int32)
    src = jnp.concatenate([edge_index[0].astype(jnp.int32), pad])
    dst = jnp.concatenate([edge_index[1].astype(jnp.int32), pad])
    batch3d = batch.astype(jnp.int32).reshape(NBLK, 1, NB)
    zeros = jnp.zeros((N, D), jnp.float32)

    e_embs = _eemb(edge_attr, W_edge)
    sc_msg = _make_sc_msg()

    h = x
    for l in range(NLAYER):
        parts = sc_msg(h, e_embs[l], src, dst, zeros)
        scale = (1.0 + eps[l]).reshape(1, 1)
        h = _mlp(parts, h, W1[l], b1[l].reshape(1, D), W2[l],
                 b2[l].reshape(1, D), scale, last=(l == NLAYER - 1))
    return _pool(batch3d, h)
